# baseline (speedup 1.0000x reference)
import jax
import jax.numpy as jnp
from jax import lax
from jax.experimental import pallas as pl
from jax.experimental.pallas import tpu as pltpu
from jax.experimental.pallas import tpu_sc as plsc

NS = 16
LANES = 16
JG = 128
NGRP = 6400


def _make_sc_call(M, D, B):
  assert D == 4 * LANES and M % 16 == 0
  half = M // 2
  nper = B // NS
  nj = nper // JG
  nl = nper // 8
  gpt = NGRP // NS
  mesh = plsc.VectorSubcoreMesh(core_axis_name="c", subcore_axis_name="s")

  def body(mem_hbm, idxr_hbm, valt8_hbm, out_hbm, s_hbm,
           idx2, tg2, sl2, w8h, ex, st, gga, ggb, sga, sgb, acc,
           gsa, gsb, osa, osb):
    c = lax.axis_index("c")
    s = lax.axis_index("s")
    gg = [gga, ggb]
    sg = [sga, sgb]
    gsem = [gsa, gsb]
    osem = [osa, osb]

    pltpu.sync_copy(idxr_hbm.at[pl.ds(s * nj, nj)], idx2)

    lo = c * half

    def _xform(gq, carry):
      v = idx2[gq >> 3, pl.ds((gq & 7) * LANES, LANES)]
      local = v - lo
      owned = (local >= 0) & (local < half)
      tg2[gq >> 3, pl.ds((gq & 7) * LANES, LANES)] = jnp.where(
          owned, local >> 3, (half >> 3) + (v & 127))
      sl2[gq >> 3, pl.ds((gq & 7) * LANES, LANES)] = v & 7
      return carry
    lax.fori_loop(0, nper // LANES, _xform, 0)

    @plsc.parallel_loop(0, JG, unroll=2)
    def _(i):
      for t in range(8):
        ex[i, pl.ds(t * LANES, LANES)] = jnp.zeros((LANES,), jnp.float32)

    def _pass(k, carry):
      for q in range(gpt // 100):
        pltpu.sync_copy(ex.at[pl.ds(0, 100)],
                        acc.at[pl.ds(s * gpt + q * 100, 100)])
      plsc.subcore_barrier()

      for hi in range(2):
        pltpu.sync_copy(
            valt8_hbm.at[k, pl.ds(s * JG + hi * (JG // 2), JG // 2)], w8h)
        for jj in range(nj // 2):
          j = hi * (nj // 2) + jj

          def _fill(q, carry):
            sv = sl2[j, pl.ds(q * LANES, LANES)]
            for l in range(LANES):
              ln = 2 * q + (l >> 3)
              ex[q * LANES + l,
                 pl.ds(sv[l] * LANES, LANES)] = w8h[
                     jj * (JG // 8) + ln, pl.ds((l & 7) * LANES, LANES)]
            return carry
          lax.fori_loop(0, JG // LANES, _fill, 0)
          pltpu.sync_copy(ex, acc.at[tg2.at[j, pl.ds(0, JG)]], add=True)

          def _clear(q, carry):
            sv = sl2[j, pl.ds(q * LANES, LANES)]
            for l in range(LANES):
              ex[q * LANES + l, pl.ds(sv[l] * LANES, LANES)] = jnp.zeros(
                  (LANES,), jnp.float32)
            return carry
          lax.fori_loop(0, JG // LANES, _clear, 0)
      plsc.subcore_barrier()

      for j in range(nj):
        for h in range(2):
          pltpu.sync_copy(acc.at[tg2.at[j, pl.ds(h * 64, 64)]],
                          w8h.at[pl.ds(0, 64)])

          def _ext(q, carry):
            sv = sl2[j, pl.ds(h * 64 + q * LANES, LANES)]
            for l in range(LANES):
              i = q * LANES + l
              st[(h * 64 + i) >> 3,
                 pl.ds((i & 7) * LANES, LANES)] = w8h[
                     i, pl.ds(sv[l] * LANES, LANES)]
            return carry
          lax.fori_loop(0, 4, _ext, 0)
        pltpu.sync_copy(
            st, s_hbm.at[c, k, pl.ds(s * nl + j * (JG // 8), JG // 8)])
      plsc.subcore_barrier()
      return carry

    lax.fori_loop(0, 4, _pass, 0)

    def _fire_g(j, dst, sem):
      def _f(q, carry):
        v = idx2[j, pl.ds(q * LANES, LANES)]
        for l in range(LANES):
          pltpu.async_copy(mem_hbm.at[pl.ds(v[l], 1)],
                           dst.at[pl.ds(q * LANES + l, 1)], sem)
        return carry
      lax.fori_loop(0, JG // LANES, _f, 0)

    def _drain_g(dst, sem):
      def _f(i, carry):
        pltpu.make_async_copy(mem_hbm.at[pl.ds(0, 1)],
                              dst.at[pl.ds(i, 1)], sem).wait()
        return carry
      lax.fori_loop(0, JG, _f, 0)

    _fire_g(0, gga, gsa)
    _fire_g(1, ggb, gsb)

    def _merge_group(j, gbuf, gsm):
      _drain_g(gbuf, gsm)
      for d in [pltpu.async_copy(
          s_hbm.at[c, kk, pl.ds(s * nl + j * (JG // 8), JG // 8)],
          w8h.at[pl.ds(kk * (JG // 8), JG // 8)], osa)
          for kk in range(4)]:
        d.wait()
      for p in range(JG // LANES):
        ob = sg[p & 1]

        @plsc.parallel_loop(0, LANES, unroll=1)
        def _(i):
          row = p * LANES + i
          for t in range(4):
            ob[i >> 3, pl.ds((i & 7) * D + t * LANES, LANES)] = (
                gbuf[row, pl.ds(t * LANES, LANES)]
                + w8h[t * (JG // 8) + (row >> 3),
                      pl.ds((row & 7) * LANES, LANES)])
        pltpu.sync_copy(
            ob, out_hbm.at[c, pl.ds(s * nl + j * (JG // 8) + p * 2, 2)])

    def _fpair(jj, carry):
      _merge_group(2 * jj, gga, gsa)

      @pl.when(jj < nj // 2 - 1)
      def _():
        _fire_g(2 * jj + 2, gga, gsa)
      _merge_group(2 * jj + 1, ggb, gsb)

      @pl.when(jj < nj // 2 - 1)
      def _():
        _fire_g(2 * jj + 3, ggb, gsb)
      return carry

    lax.fori_loop(0, nj // 2, _fpair, 0)

  call = pl.kernel(
      body,
      out_type=(jax.ShapeDtypeStruct((2, B // 8, 8 * D), jnp.float32),
                jax.ShapeDtypeStruct((2, 4, B // 8, 8 * LANES),
                                     jnp.float32)),
      mesh=mesh,
      scratch_types=[
          pltpu.VMEM((nj, JG), jnp.int32),
          pltpu.VMEM((nj, JG), jnp.int32),
          pltpu.VMEM((nj, JG), jnp.int32),
          pltpu.VMEM((JG // 2, 8 * LANES), jnp.float32),
          pltpu.VMEM((JG, 8 * LANES), jnp.float32),
          pltpu.VMEM((JG // 8, 8 * LANES), jnp.float32),
          pltpu.VMEM((JG, D), jnp.float32),
          pltpu.VMEM((JG, D), jnp.float32),
          pltpu.VMEM((2, 8 * D), jnp.float32),
          pltpu.VMEM((2, 8 * D), jnp.float32),
          pltpu.VMEM_SHARED((NGRP, 8 * LANES), jnp.float32),
          pltpu.SemaphoreType.DMA,
          pltpu.SemaphoreType.DMA,
          pltpu.SemaphoreType.DMA,
          pltpu.SemaphoreType.DMA,
      ],
  )
  return call


@jax.jit
def kernel(mem, idx, val):
  M, D = mem.shape
  B = idx.shape[0]
  idx32 = idx.astype(jnp.int32)
  idxr = idx32.reshape(B // JG, JG)
  valt8 = val.reshape(B, 4, LANES).transpose(1, 0, 2).reshape(
      4, B // 8, 8 * LANES)
  outp, _ = _make_sc_call(M, D, B)(mem, idxr, valt8)
  both = outp.reshape(2, B, D)
  return jnp.where((idx32 < M // 2)[:, None], both[0], both[1])


# speedup vs baseline: 1.0299x; 1.0299x over previous
import jax
import jax.numpy as jnp
from jax import lax
from jax.experimental import pallas as pl
from jax.experimental.pallas import tpu as pltpu
from jax.experimental.pallas import tpu_sc as plsc

NS = 16
LANES = 16
JG = 128
NGRP = 6400


def _make_sc_call(M, D, B):
  assert D == 4 * LANES and M % 16 == 0
  half = M // 2
  nper = B // NS
  nj = nper // JG
  nl = nper // 8
  gpt = NGRP // NS
  mesh = plsc.VectorSubcoreMesh(core_axis_name="c", subcore_axis_name="s")

  def body(mem_hbm, idxr_hbm, valt8_hbm, out_hbm, s_hbm,
           idx2, tg2, sl2, w8h, ex, sta, stb, sgo, gga, ggb, acc,
           gsa, gsb, osa, osb):
    c = lax.axis_index("c")
    s = lax.axis_index("s")
    st2 = [sta, stb]
    osem = [osa, osb]

    pltpu.sync_copy(idxr_hbm.at[pl.ds(s * nj, nj)], idx2)

    lo = c * half

    def _xform(gq, carry):
      v = idx2[gq >> 3, pl.ds((gq & 7) * LANES, LANES)]
      local = v - lo
      owned = (local >= 0) & (local < half)
      tg2[gq >> 3, pl.ds((gq & 7) * LANES, LANES)] = jnp.where(
          owned, local >> 3, (half >> 3) + (v & 127))
      sl2[gq >> 3, pl.ds((gq & 7) * LANES, LANES)] = v & 7
      return carry
    lax.fori_loop(0, nper // LANES, _xform, 0)

    @plsc.parallel_loop(0, JG, unroll=2)
    def _(i):
      for t in range(8):
        ex[i, pl.ds(t * LANES, LANES)] = jnp.zeros((LANES,), jnp.float32)

    def _pass(k, carry):
      for q in range(gpt // 100):
        pltpu.sync_copy(ex.at[pl.ds(0, 100)],
                        acc.at[pl.ds(s * gpt + q * 100, 100)])
      plsc.subcore_barrier()

      for hi in range(2):
        pltpu.sync_copy(
            valt8_hbm.at[k, pl.ds(s * JG + hi * (JG // 2), JG // 2)], w8h)
        for jj in range(nj // 2):
          j = hi * (nj // 2) + jj

          def _fill(q, carry):
            sv = sl2[j, pl.ds(q * LANES, LANES)]
            for l in range(LANES):
              ln = 2 * q + (l >> 3)
              ex[q * LANES + l,
                 pl.ds(sv[l] * LANES, LANES)] = w8h[
                     jj * (JG // 8) + ln, pl.ds((l & 7) * LANES, LANES)]
            return carry
          lax.fori_loop(0, JG // LANES, _fill, 0)
          pltpu.sync_copy(ex, acc.at[tg2.at[j, pl.ds(0, JG)]], add=True)

          def _clear(q, carry):
            sv = sl2[j, pl.ds(q * LANES, LANES)]
            for l in range(LANES):
              ex[q * LANES + l, pl.ds(sv[l] * LANES, LANES)] = jnp.zeros(
                  (LANES,), jnp.float32)
            return carry
          lax.fori_loop(0, JG // LANES, _clear, 0)
      plsc.subcore_barrier()

      sdescs = [None, None]
      for j in range(nj):
        stj = st2[j & 1]
        if sdescs[j & 1] is not None:
          sdescs[j & 1].wait()
        for h in range(2):
          pltpu.sync_copy(acc.at[tg2.at[j, pl.ds(h * 64, 64)]],
                          w8h.at[pl.ds(0, 64)])

          def _ext(q, carry):
            sv = sl2[j, pl.ds(h * 64 + q * LANES, LANES)]
            for l in range(LANES):
              i = q * LANES + l
              stj[(h * 64 + i) >> 3,
                  pl.ds((i & 7) * LANES, LANES)] = w8h[
                      i, pl.ds(sv[l] * LANES, LANES)]
            return carry
          lax.fori_loop(0, 4, _ext, 0)
        sdescs[j & 1] = pltpu.async_copy(
            stj, s_hbm.at[c, k, pl.ds(s * nl + j * (JG // 8), JG // 8)],
            osem[j & 1])
      for d in sdescs:
        d.wait()
      plsc.subcore_barrier()
      return carry

    lax.fori_loop(0, 4, _pass, 0)

    def _fire_g(j, dst, sem):
      def _f(q, carry):
        v = idx2[j, pl.ds(q * LANES, LANES)]
        for l in range(LANES):
          pltpu.async_copy(mem_hbm.at[pl.ds(v[l], 1)],
                           dst.at[pl.ds(q * LANES + l, 1)], sem)
        return carry
      lax.fori_loop(0, JG // LANES, _f, 0)

    def _drain_g(dst, sem):
      def _f(i, carry):
        pltpu.make_async_copy(mem_hbm.at[pl.ds(0, 1)],
                              dst.at[pl.ds(i, 1)], sem).wait()
        return carry
      lax.fori_loop(0, JG, _f, 0)

    _fire_g(0, gga, gsa)
    _fire_g(1, ggb, gsb)

    def _merge_group(j, gbuf, gsm):
      _drain_g(gbuf, gsm)
      for d in [pltpu.async_copy(
          s_hbm.at[c, kk, pl.ds(s * nl + j * (JG // 8), JG // 8)],
          w8h.at[pl.ds(kk * (JG // 8), JG // 8)], osa)
          for kk in range(4)]:
        d.wait()
      @plsc.parallel_loop(0, JG, unroll=1)
      def _(row):
        for t in range(4):
          sgo[row >> 3, pl.ds((row & 7) * D + t * LANES, LANES)] = (
              gbuf[row, pl.ds(t * LANES, LANES)]
              + w8h[t * (JG // 8) + (row >> 3),
                    pl.ds((row & 7) * LANES, LANES)])
      pltpu.sync_copy(
          sgo, out_hbm.at[c, pl.ds(s * nl + j * (JG // 8), JG // 8)])

    def _fpair(jj, carry):
      _merge_group(2 * jj, gga, gsa)

      @pl.when(jj < nj // 2 - 1)
      def _():
        _fire_g(2 * jj + 2, gga, gsa)
      _merge_group(2 * jj + 1, ggb, gsb)

      @pl.when(jj < nj // 2 - 1)
      def _():
        _fire_g(2 * jj + 3, ggb, gsb)
      return carry

    lax.fori_loop(0, nj // 2, _fpair, 0)

  call = pl.kernel(
      body,
      out_type=(jax.ShapeDtypeStruct((2, B // 8, 8 * D), jnp.float32),
                jax.ShapeDtypeStruct((2, 4, B // 8, 8 * LANES),
                                     jnp.float32)),
      mesh=mesh,
      scratch_types=[
          pltpu.VMEM((nj, JG), jnp.int32),
          pltpu.VMEM((nj, JG), jnp.int32),
          pltpu.VMEM((nj, JG), jnp.int32),
          pltpu.VMEM((JG // 2, 8 * LANES), jnp.float32),
          pltpu.VMEM((JG, 8 * LANES), jnp.float32),
          pltpu.VMEM((JG // 8, 8 * LANES), jnp.float32),
          pltpu.VMEM((JG // 8, 8 * LANES), jnp.float32),
          pltpu.VMEM((JG // 8, 8 * D), jnp.float32),
          pltpu.VMEM((JG, D), jnp.float32),
          pltpu.VMEM((JG, D), jnp.float32),
          pltpu.VMEM_SHARED((NGRP, 8 * LANES), jnp.float32),
          pltpu.SemaphoreType.DMA,
          pltpu.SemaphoreType.DMA,
          pltpu.SemaphoreType.DMA,
          pltpu.SemaphoreType.DMA,
      ],
  )
  return call


@jax.jit
def kernel(mem, idx, val):
  M, D = mem.shape
  B = idx.shape[0]
  idx32 = idx.astype(jnp.int32)
  idxr = idx32.reshape(B // JG, JG)
  valt8 = val.reshape(B, 4, LANES).transpose(1, 0, 2).reshape(
      4, B // 8, 8 * LANES)
  outp, _ = _make_sc_call(M, D, B)(mem, idxr, valt8)
  both = outp.reshape(2, B, D)
  return jnp.where((idx32 < M // 2)[:, None], both[0], both[1])
